# baseline (speedup 1.0000x reference)
import functools

import jax
import jax.numpy as jnp
from jax import lax
from jax.experimental import pallas as pl
from jax.experimental.pallas import tpu as pltpu
from jax.experimental.pallas import tpu_sc as plsc

N = 10000
E = 320000
D = 128
R = 8

NC = 2
NS = 16
NW = NC * NS
EPW = E // NW
CHUNK = 128
NCHUNK = 80
EPW_PAD = NCHUNK * CHUNK
NBUF = 2
NDUM = 8
RPT = N // NS
ZREP = RPT // CHUNK
ZREM = RPT - ZREP * CHUNK


def _table_body(x_ref, rel_ref, out_ref):
    r = pl.program_id(0)
    out_ref[...] = x_ref[...] * rel_ref[r, :][None, :]


def _build_table(x, rel_emb):
    return pl.pallas_call(
        _table_body,
        grid=(R,),
        in_specs=[
            pl.BlockSpec((N, D), lambda r: (0, 0)),
            pl.BlockSpec((R, D), lambda r: (0, 0)),
        ],
        out_specs=pl.BlockSpec((N, D), lambda r: (r, 0)),
        out_shape=jax.ShapeDtypeStruct((R * N, D), jnp.float32),
    )(x, rel_emb)


def _ci_body(src_ref, et_ref, out_ref):
    out_ref[...] = et_ref[...] * N + src_ref[...]


def _build_ci(src2, et2):
    return pl.pallas_call(
        _ci_body,
        out_shape=jax.ShapeDtypeStruct(src2.shape, jnp.int32),
    )(src2, et2)


def _sc_agg_body(table_hbm, ci_hbm, dst_hbm, out_hbm, ci_v, dst_v, rows_v,
                 agg_sh, gsems, dsems):
    cid = lax.axis_index("c")
    sid = lax.axis_index("s")
    wid = sid * NC + cid

    pltpu.sync_copy(ci_hbm.at[wid], ci_v)

    def _zrow(r, c):
        for dd in range(D // 16):
            rows_v[0, r, pl.ds(dd * 16, 16)] = jnp.zeros((16,), jnp.float32)
        return c

    lax.fori_loop(0, CHUNK, _zrow, 0)
    for j in range(ZREP):
        pltpu.sync_copy(rows_v.at[0],
                        agg_sh.at[pl.ds(sid * RPT + j * CHUNK, CHUNK)])
    if ZREM:
        pltpu.sync_copy(rows_v.at[0, pl.ds(0, ZREM)],
                        agg_sh.at[pl.ds(sid * RPT + ZREP * CHUNK, ZREM)])
    @pl.when(sid == 0)
    def _():
        pltpu.sync_copy(rows_v.at[0, pl.ds(0, NDUM)],
                        agg_sh.at[pl.ds(N, NDUM)])

    plsc.subcore_barrier()

    def _gather(k, b):
        pltpu.async_copy(table_hbm.at[ci_v.at[k]], rows_v.at[b], gsems.at[b])
        pltpu.async_copy(dst_hbm.at[wid, k], dst_v.at[b], dsems.at[b])

    def _wait(b):
        pltpu.make_async_copy(table_hbm.at[pl.ds(0, CHUNK)], rows_v.at[b],
                              gsems.at[b]).wait()
        pltpu.make_async_copy(dst_hbm.at[0, 0], dst_v.at[b],
                              dsems.at[b]).wait()

    for b in range(NBUF):
        _gather(b, b)

    def _edge_group(g, c):
        k0 = g * NBUF
        for b in range(NBUF):
            _wait(b)
            pltpu.sync_copy(rows_v.at[b], agg_sh.at[dst_v.at[b]], add=True)
            _gather(k0 + b + NBUF, b)
        return c

    lax.fori_loop(0, NCHUNK // NBUF, _edge_group, 0)
    for b in range(NBUF):
        _wait(b)
    plsc.subcore_barrier()

    pltpu.sync_copy(agg_sh.at[pl.ds(sid * RPT, RPT)], out_hbm.at[cid, sid])


_sc_agg = functools.partial(
    pl.kernel,
    out_type=jax.ShapeDtypeStruct((NC, NS, RPT, D), jnp.float32),
    mesh=plsc.VectorSubcoreMesh(core_axis_name="c", subcore_axis_name="s"),
    scratch_types=[
        pltpu.VMEM((NCHUNK + NBUF, CHUNK), jnp.int32),
        pltpu.VMEM((NBUF, CHUNK), jnp.int32),
        pltpu.VMEM((NBUF, CHUNK, D), jnp.float32),
        pltpu.VMEM_SHARED((N + NDUM, D), jnp.float32),
        pltpu.SemaphoreType.DMA((NBUF,)),
        pltpu.SemaphoreType.DMA((NBUF,)),
    ],
)(_sc_agg_body)


def _out_body(agg_ref, x_ref, w_ref, ws_ref, out_ref):
    a = agg_ref[0] + agg_ref[1]
    out_ref[...] = (
        jnp.dot(a, w_ref[...], preferred_element_type=jnp.float32)
        + jnp.dot(x_ref[...], ws_ref[...], preferred_element_type=jnp.float32)
    )


def _combine(agg2, x, W, W_self):
    NB = 2000
    return pl.pallas_call(
        _out_body,
        grid=(N // NB,),
        in_specs=[
            pl.BlockSpec((NC, NB, D), lambda i: (0, i, 0)),
            pl.BlockSpec((NB, D), lambda i: (i, 0)),
            pl.BlockSpec((D, D), lambda i: (0, 0)),
            pl.BlockSpec((D, D), lambda i: (0, 0)),
        ],
        out_specs=pl.BlockSpec((NB, D), lambda i: (i, 0)),
        out_shape=jax.ShapeDtypeStruct((N, D), jnp.float32),
    )(agg2, x, W, W_self)


def kernel(x, edge_index, edge_type, rel_emb, W, W_self):
    src = edge_index[0].astype(jnp.int32)
    dst = edge_index[1].astype(jnp.int32)
    et = edge_type.astype(jnp.int32)

    table = _build_table(x, rel_emb)
    ci2 = _build_ci(src.reshape(E // D, D), et.reshape(E // D, D))

    pad = EPW_PAD - EPW
    ciw = ci2.reshape(NW, EPW)
    ci_sl = jnp.concatenate(
        [ciw, jnp.zeros((NW, pad + NBUF * CHUNK), jnp.int32)], axis=1
    ).reshape(NW, NCHUNK + NBUF, CHUNK)
    dummy = N + (jnp.arange(NW, dtype=jnp.int32) % NDUM)
    dst_sl = jnp.concatenate(
        [dst.reshape(NW, EPW),
         jnp.broadcast_to(dummy[:, None], (NW, pad + NBUF * CHUNK))], axis=1
    ).reshape(NW, NCHUNK + NBUF, CHUNK)

    agg2 = _sc_agg(table, ci_sl, dst_sl).reshape(NC, N, D)
    return _combine(agg2, x, W, W_self)


# speedup vs baseline: 3.1149x; 3.1149x over previous
import functools

import jax
import jax.numpy as jnp
from jax import lax
from jax.experimental import pallas as pl
from jax.experimental.pallas import tpu as pltpu
from jax.experimental.pallas import tpu_sc as plsc

N = 10000
E = 320000
D = 128
R = 8

NC = 2
NS = 16
NW = NC * NS
EPW = E // NW
CHUNK = 125
NCHUNK = EPW // CHUNK
RPT = N // NS
ZREP = RPT // CHUNK


def _table_body(x_ref, rel_ref, out_ref):
    r = pl.program_id(0)
    out_ref[...] = x_ref[...] * rel_ref[r, :][None, :]


def _build_table(x, rel_emb):
    return pl.pallas_call(
        _table_body,
        grid=(R,),
        in_specs=[
            pl.BlockSpec((N, D), lambda r: (0, 0)),
            pl.BlockSpec((R, D), lambda r: (0, 0)),
        ],
        out_specs=pl.BlockSpec((N, D), lambda r: (r, 0)),
        out_shape=jax.ShapeDtypeStruct((R * N, D), jnp.float32),
    )(x, rel_emb)


def _ci_body(src_ref, et_ref, out_ref):
    out_ref[...] = et_ref[...] * N + src_ref[...]


def _build_ci(src2, et2):
    return pl.pallas_call(
        _ci_body,
        out_shape=jax.ShapeDtypeStruct(src2.shape, jnp.int32),
    )(src2, et2)


def _sc_agg_body(table_hbm, ci_hbm, dst_hbm, out_hbm, ci_v, dst_v, rows_v,
                 agg_sh):
    cid = lax.axis_index("c")
    sid = lax.axis_index("s")
    wid = sid * NC + cid

    pltpu.sync_copy(ci_hbm.at[wid], ci_v)
    pltpu.sync_copy(dst_hbm.at[wid], dst_v)

    def _zrow(r, c):
        for dd in range(D // 16):
            rows_v[r, pl.ds(dd * 16, 16)] = jnp.zeros((16,), jnp.float32)
        return c

    lax.fori_loop(0, CHUNK, _zrow, 0)
    for j in range(ZREP):
        pltpu.sync_copy(rows_v,
                        agg_sh.at[pl.ds(sid * RPT + j * CHUNK, CHUNK)])

    plsc.subcore_barrier()

    def _chunk(k, c):
        pltpu.sync_copy(table_hbm.at[ci_v.at[k]], rows_v)
        pltpu.sync_copy(rows_v, agg_sh.at[dst_v.at[k]], add=True)
        return c

    lax.fori_loop(0, NCHUNK, _chunk, 0)
    plsc.subcore_barrier()

    pltpu.sync_copy(agg_sh.at[pl.ds(sid * RPT, RPT)], out_hbm.at[cid, sid])


_sc_agg = functools.partial(
    pl.kernel,
    out_type=jax.ShapeDtypeStruct((NC, NS, RPT, D), jnp.float32),
    mesh=plsc.VectorSubcoreMesh(core_axis_name="c", subcore_axis_name="s"),
    scratch_types=[
        pltpu.VMEM((NCHUNK, CHUNK), jnp.int32),
        pltpu.VMEM((NCHUNK, CHUNK), jnp.int32),
        pltpu.VMEM((CHUNK, D), jnp.float32),
        pltpu.VMEM_SHARED((N, D), jnp.float32),
    ],
)(_sc_agg_body)


def _out_body(agg_ref, x_ref, w_ref, ws_ref, out_ref):
    a = agg_ref[0] + agg_ref[1]
    out_ref[...] = (
        jnp.dot(a, w_ref[...], preferred_element_type=jnp.float32)
        + jnp.dot(x_ref[...], ws_ref[...], preferred_element_type=jnp.float32)
    )


def _combine(agg2, x, W, W_self):
    NB = 2000
    return pl.pallas_call(
        _out_body,
        grid=(N // NB,),
        in_specs=[
            pl.BlockSpec((NC, NB, D), lambda i: (0, i, 0)),
            pl.BlockSpec((NB, D), lambda i: (i, 0)),
            pl.BlockSpec((D, D), lambda i: (0, 0)),
            pl.BlockSpec((D, D), lambda i: (0, 0)),
        ],
        out_specs=pl.BlockSpec((NB, D), lambda i: (i, 0)),
        out_shape=jax.ShapeDtypeStruct((N, D), jnp.float32),
    )(agg2, x, W, W_self)


def kernel(x, edge_index, edge_type, rel_emb, W, W_self):
    src = edge_index[0].astype(jnp.int32)
    dst = edge_index[1].astype(jnp.int32)
    et = edge_type.astype(jnp.int32)

    table = _build_table(x, rel_emb)
    ci2 = _build_ci(src.reshape(E // D, D), et.reshape(E // D, D))

    ci_sl = ci2.reshape(NW, NCHUNK, CHUNK)
    dst_sl = dst.reshape(NW, NCHUNK, CHUNK)

    agg2 = _sc_agg(table, ci_sl, dst_sl).reshape(NC, N, D)
    return _combine(agg2, x, W, W_self)


# speedup vs baseline: 3.7858x; 1.2154x over previous
import functools

import jax
import jax.numpy as jnp
from jax import lax
from jax.experimental import pallas as pl
from jax.experimental.pallas import tpu as pltpu
from jax.experimental.pallas import tpu_sc as plsc

N = 10000
E = 320000
D = 128
R = 8

NC = 2
NS = 16
NW = NC * NS
EPW = E // NW
CHUNK = 125
NCHUNK = EPW // CHUNK
RPT = N // NS
ZREP = RPT // CHUNK


def _table_body(x_ref, rel_ref, out_ref):
    r = pl.program_id(0)
    out_ref[...] = x_ref[...] * rel_ref[r, :][None, :]


def _build_table(x, rel_emb):
    return pl.pallas_call(
        _table_body,
        grid=(R,),
        in_specs=[
            pl.BlockSpec((N, D), lambda r: (0, 0)),
            pl.BlockSpec((R, D), lambda r: (0, 0)),
        ],
        out_specs=pl.BlockSpec((N, D), lambda r: (r, 0)),
        out_shape=jax.ShapeDtypeStruct((R * N, D), jnp.float32),
    )(x, rel_emb)


def _ci_body(src_ref, et_ref, out_ref):
    out_ref[...] = et_ref[...] * N + src_ref[...]


def _build_ci(src2, et2):
    return pl.pallas_call(
        _ci_body,
        out_shape=jax.ShapeDtypeStruct(src2.shape, jnp.int32),
    )(src2, et2)


NPH = 2
CPP = NCHUNK // NPH
NBUF = 2


def _sc_agg_body(table_hbm, ci_hbm, dst_hbm, dum_hbm, out_hbm, ci_v, dst_v,
                 rows_v, agg_sh, ssems):
    cid = lax.axis_index("c")
    sid = lax.axis_index("s")
    wid = sid * NC + cid

    def _zrow(r, c):
        for dd in range(D // 16):
            rows_v[0, r, pl.ds(dd * 16, 16)] = jnp.zeros((16,), jnp.float32)
        return c

    lax.fori_loop(0, CHUNK, _zrow, 0)
    for j in range(ZREP):
        pltpu.sync_copy(rows_v.at[0],
                        agg_sh.at[pl.ds(sid * RPT + j * CHUNK, CHUNK)])

    plsc.subcore_barrier()

    def _drain(b):
        pltpu.make_async_copy(dum_hbm, rows_v.at[b], ssems.at[b]).wait()

    for p in range(NPH):
        pltpu.sync_copy(ci_hbm.at[wid, pl.ds(p * CPP, CPP)], ci_v)
        pltpu.sync_copy(dst_hbm.at[wid, pl.ds(p * CPP, CPP)], dst_v)

        for b in range(NBUF):
            pltpu.sync_copy(table_hbm.at[ci_v.at[b]], rows_v.at[b])
            pltpu.async_copy(rows_v.at[b], agg_sh.at[dst_v.at[b]],
                             ssems.at[b], add=True)

        def _group(g, c):
            for b in range(NBUF):
                k = g * NBUF + b
                _drain(b)
                pltpu.sync_copy(table_hbm.at[ci_v.at[k]], rows_v.at[b])
                pltpu.async_copy(rows_v.at[b], agg_sh.at[dst_v.at[k]],
                                 ssems.at[b], add=True)
            return c

        lax.fori_loop(1, CPP // NBUF, _group, 0)
        for b in range(NBUF):
            _drain(b)

    plsc.subcore_barrier()

    pltpu.sync_copy(agg_sh.at[pl.ds(sid * RPT, RPT)], out_hbm.at[cid, sid])


_sc_agg = functools.partial(
    pl.kernel,
    out_type=jax.ShapeDtypeStruct((NC, NS, RPT, D), jnp.float32),
    mesh=plsc.VectorSubcoreMesh(core_axis_name="c", subcore_axis_name="s"),
    scratch_types=[
        pltpu.VMEM((CPP, CHUNK), jnp.int32),
        pltpu.VMEM((CPP, CHUNK), jnp.int32),
        pltpu.VMEM((NBUF, CHUNK, D), jnp.float32),
        pltpu.VMEM_SHARED((N, D), jnp.float32),
        pltpu.SemaphoreType.DMA((NBUF,)),
    ],
)(_sc_agg_body)


def _out_body(agg_ref, x_ref, w_ref, ws_ref, out_ref):
    a = agg_ref[0] + agg_ref[1]
    out_ref[...] = (
        jnp.dot(a, w_ref[...], preferred_element_type=jnp.float32)
        + jnp.dot(x_ref[...], ws_ref[...], preferred_element_type=jnp.float32)
    )


def _combine(agg2, x, W, W_self):
    NB = 2000
    return pl.pallas_call(
        _out_body,
        grid=(N // NB,),
        in_specs=[
            pl.BlockSpec((NC, NB, D), lambda i: (0, i, 0)),
            pl.BlockSpec((NB, D), lambda i: (i, 0)),
            pl.BlockSpec((D, D), lambda i: (0, 0)),
            pl.BlockSpec((D, D), lambda i: (0, 0)),
        ],
        out_specs=pl.BlockSpec((NB, D), lambda i: (i, 0)),
        out_shape=jax.ShapeDtypeStruct((N, D), jnp.float32),
    )(agg2, x, W, W_self)


def kernel(x, edge_index, edge_type, rel_emb, W, W_self):
    src = edge_index[0].astype(jnp.int32)
    dst = edge_index[1].astype(jnp.int32)
    et = edge_type.astype(jnp.int32)

    table = _build_table(x, rel_emb)
    ci2 = _build_ci(src.reshape(E // D, D), et.reshape(E // D, D))

    ci_sl = ci2.reshape(NW, NCHUNK, CHUNK)
    dst_sl = dst.reshape(NW, NCHUNK, CHUNK)

    dum = jnp.zeros((CHUNK, D), jnp.float32)
    agg2 = _sc_agg(table, ci_sl, dst_sl, dum).reshape(NC, N, D)
    return _combine(agg2, x, W, W_self)


# speedup vs baseline: 4.2041x; 1.1105x over previous
import functools

import jax
import jax.numpy as jnp
from jax import lax
from jax.experimental import pallas as pl
from jax.experimental.pallas import tpu as pltpu
from jax.experimental.pallas import tpu_sc as plsc

N = 10000
E = 320000
D = 128
R = 8

NC = 2
NS = 16
NW = NC * NS
EPW = E // NW
CHUNK = 125
NCHUNK = EPW // CHUNK
RPT = N // NS
ZREP = RPT // CHUNK


def _table_body(x_ref, rel_ref, out_ref):
    r = pl.program_id(0)
    out_ref[...] = x_ref[...] * rel_ref[r, :][None, :]


def _build_table(x, rel_emb):
    return pl.pallas_call(
        _table_body,
        grid=(R,),
        in_specs=[
            pl.BlockSpec((N, D), lambda r: (0, 0)),
            pl.BlockSpec((R, D), lambda r: (0, 0)),
        ],
        out_specs=pl.BlockSpec((N, D), lambda r: (r, 0)),
        out_shape=jax.ShapeDtypeStruct((R * N, D), jnp.float32),
    )(x, rel_emb)


def _ci_body(src_ref, et_ref, out_ref):
    out_ref[...] = et_ref[...] * N + src_ref[...]


def _build_ci(src2, et2):
    return pl.pallas_call(
        _ci_body,
        out_shape=jax.ShapeDtypeStruct(src2.shape, jnp.int32),
    )(src2, et2)


NPH = 2
CPP = NCHUNK // NPH
NBUF = 2


def _sc_agg_body(table_hbm, ci_hbm, dst_hbm, dum_hbm, out_hbm, ci_v, dst_v,
                 rows_v, agg_sh, gsems):
    cid = lax.axis_index("c")
    sid = lax.axis_index("s")
    wid = sid * NC + cid

    def _zrow(r, c):
        for dd in range(D // 16):
            rows_v[0, r, pl.ds(dd * 16, 16)] = jnp.zeros((16,), jnp.float32)
        return c

    lax.fori_loop(0, CHUNK, _zrow, 0)
    for j in range(ZREP):
        pltpu.sync_copy(rows_v.at[0],
                        agg_sh.at[pl.ds(sid * RPT + j * CHUNK, CHUNK)])

    plsc.subcore_barrier()

    def _drain(b):
        pltpu.make_async_copy(dum_hbm, rows_v.at[b], gsems.at[b]).wait()

    for p in range(NPH):
        pltpu.sync_copy(ci_hbm.at[wid, pl.ds(p * CPP, CPP)], ci_v)
        pltpu.sync_copy(dst_hbm.at[wid, pl.ds(p * CPP, CPP)], dst_v)

        for b in range(NBUF):
            pltpu.async_copy(table_hbm.at[ci_v.at[b]], rows_v.at[b],
                             gsems.at[b])

        def _group(g, c):
            for b in range(NBUF):
                k = g * NBUF + b
                _drain(b)
                pltpu.sync_copy(rows_v.at[b], agg_sh.at[dst_v.at[k]],
                                add=True)
                pltpu.async_copy(table_hbm.at[ci_v.at[k + NBUF]],
                                 rows_v.at[b], gsems.at[b])
            return c

        lax.fori_loop(0, CPP // NBUF - 1, _group, 0)
        for b in range(NBUF):
            _drain(b)
            pltpu.sync_copy(rows_v.at[b],
                            agg_sh.at[dst_v.at[CPP - NBUF + b]], add=True)

    plsc.subcore_barrier()

    pltpu.sync_copy(agg_sh.at[pl.ds(sid * RPT, RPT)], out_hbm.at[cid, sid])


_sc_agg = functools.partial(
    pl.kernel,
    out_type=jax.ShapeDtypeStruct((NC, NS, RPT, D), jnp.float32),
    mesh=plsc.VectorSubcoreMesh(core_axis_name="c", subcore_axis_name="s"),
    scratch_types=[
        pltpu.VMEM((CPP, CHUNK), jnp.int32),
        pltpu.VMEM((CPP, CHUNK), jnp.int32),
        pltpu.VMEM((NBUF, CHUNK, D), jnp.float32),
        pltpu.VMEM_SHARED((N, D), jnp.float32),
        pltpu.SemaphoreType.DMA((NBUF,)),
    ],
)(_sc_agg_body)


def _out_body(agg_ref, x_ref, w_ref, ws_ref, out_ref):
    a = agg_ref[0] + agg_ref[1]
    out_ref[...] = (
        jnp.dot(a, w_ref[...], preferred_element_type=jnp.float32)
        + jnp.dot(x_ref[...], ws_ref[...], preferred_element_type=jnp.float32)
    )


def _combine(agg2, x, W, W_self):
    NB = 2000
    return pl.pallas_call(
        _out_body,
        grid=(N // NB,),
        in_specs=[
            pl.BlockSpec((NC, NB, D), lambda i: (0, i, 0)),
            pl.BlockSpec((NB, D), lambda i: (i, 0)),
            pl.BlockSpec((D, D), lambda i: (0, 0)),
            pl.BlockSpec((D, D), lambda i: (0, 0)),
        ],
        out_specs=pl.BlockSpec((NB, D), lambda i: (i, 0)),
        out_shape=jax.ShapeDtypeStruct((N, D), jnp.float32),
    )(agg2, x, W, W_self)


def kernel(x, edge_index, edge_type, rel_emb, W, W_self):
    src = edge_index[0].astype(jnp.int32)
    dst = edge_index[1].astype(jnp.int32)
    et = edge_type.astype(jnp.int32)

    table = _build_table(x, rel_emb)
    ci2 = _build_ci(src.reshape(E // D, D), et.reshape(E // D, D))

    ci_sl = ci2.reshape(NW, NCHUNK, CHUNK)
    dst_sl = dst.reshape(NW, NCHUNK, CHUNK)

    dum = jnp.zeros((CHUNK, D), jnp.float32)
    agg2 = _sc_agg(table, ci_sl, dst_sl, dum).reshape(NC, N, D)
    return _combine(agg2, x, W, W_self)


# speedup vs baseline: 4.2989x; 1.0226x over previous
import functools

import jax
import jax.numpy as jnp
from jax import lax
from jax.experimental import pallas as pl
from jax.experimental.pallas import tpu as pltpu
from jax.experimental.pallas import tpu_sc as plsc

N = 10000
E = 320000
D = 128
R = 8

NC = 2
NS = 16
NW = NC * NS
EPW = E // NW
CHUNK = 100
NCHUNK = EPW // CHUNK
RPT = N // NS
ZREP = RPT // CHUNK
ZREM = RPT - ZREP * CHUNK


def _table_body(x_ref, rel_ref, out_ref):
    r = pl.program_id(0)
    out_ref[...] = x_ref[...] * rel_ref[r, :][None, :]


def _build_table(x, rel_emb):
    return pl.pallas_call(
        _table_body,
        grid=(R,),
        in_specs=[
            pl.BlockSpec((N, D), lambda r: (0, 0)),
            pl.BlockSpec((R, D), lambda r: (0, 0)),
        ],
        out_specs=pl.BlockSpec((N, D), lambda r: (r, 0)),
        out_shape=jax.ShapeDtypeStruct((R * N, D), jnp.float32),
    )(x, rel_emb)


def _ci_body(src_ref, et_ref, out_ref):
    out_ref[...] = et_ref[...] * N + src_ref[...]


def _build_ci(src2, et2):
    return pl.pallas_call(
        _ci_body,
        out_shape=jax.ShapeDtypeStruct(src2.shape, jnp.int32),
    )(src2, et2)


NPH = 5
CPP = NCHUNK // NPH
NBUF = 3
NGRP = (CPP - NBUF) // NBUF


def _sc_agg_body(table_hbm, ci_hbm, dst_hbm, dum_hbm, out_hbm, ci_v, dst_v,
                 rows_v, agg_sh, gsems):
    cid = lax.axis_index("c")
    sid = lax.axis_index("s")
    wid = sid * NC + cid

    def _zrow(r, c):
        for dd in range(D // 16):
            rows_v[0, r, pl.ds(dd * 16, 16)] = jnp.zeros((16,), jnp.float32)
        return c

    lax.fori_loop(0, CHUNK, _zrow, 0)
    for j in range(ZREP):
        pltpu.sync_copy(rows_v.at[0],
                        agg_sh.at[pl.ds(sid * RPT + j * CHUNK, CHUNK)])
    if ZREM:
        pltpu.sync_copy(rows_v.at[0, pl.ds(0, ZREM)],
                        agg_sh.at[pl.ds(sid * RPT + ZREP * CHUNK, ZREM)])

    plsc.subcore_barrier()

    def _drain(b):
        pltpu.make_async_copy(dum_hbm, rows_v.at[b], gsems.at[b]).wait()

    for p in range(NPH):
        pltpu.sync_copy(ci_hbm.at[wid, p], ci_v)
        pltpu.sync_copy(dst_hbm.at[wid, p], dst_v)

        for b in range(NBUF):
            pltpu.async_copy(table_hbm.at[ci_v.at[b]], rows_v.at[b],
                             gsems.at[b])

        def _group(g, c):
            for b in range(NBUF):
                k = g * NBUF + b
                _drain(b)
                pltpu.sync_copy(rows_v.at[b], agg_sh.at[dst_v.at[k]],
                                add=True)
                pltpu.async_copy(table_hbm.at[ci_v.at[k + NBUF]],
                                 rows_v.at[b], gsems.at[b])
            return c

        lax.fori_loop(0, NGRP, _group, 0)
        for k in range(NGRP * NBUF, CPP):
            b = k % NBUF
            _drain(b)
            pltpu.sync_copy(rows_v.at[b], agg_sh.at[dst_v.at[k]], add=True)
            if k + NBUF < CPP:
                pltpu.async_copy(table_hbm.at[ci_v.at[k + NBUF]],
                                 rows_v.at[b], gsems.at[b])

    plsc.subcore_barrier()

    pltpu.sync_copy(agg_sh.at[pl.ds(sid * RPT, RPT)], out_hbm.at[cid, sid])


_sc_agg = functools.partial(
    pl.kernel,
    out_type=jax.ShapeDtypeStruct((NC, NS, RPT, D), jnp.float32),
    mesh=plsc.VectorSubcoreMesh(core_axis_name="c", subcore_axis_name="s"),
    scratch_types=[
        pltpu.VMEM((CPP, CHUNK), jnp.int32),
        pltpu.VMEM((CPP, CHUNK), jnp.int32),
        pltpu.VMEM((NBUF, CHUNK, D), jnp.float32),
        pltpu.VMEM_SHARED((N, D), jnp.float32),
        pltpu.SemaphoreType.DMA((NBUF,)),
    ],
)(_sc_agg_body)


def _out_body(agg_ref, x_ref, w_ref, ws_ref, out_ref):
    a = agg_ref[0] + agg_ref[1]
    out_ref[...] = (
        jnp.dot(a, w_ref[...], preferred_element_type=jnp.float32)
        + jnp.dot(x_ref[...], ws_ref[...], preferred_element_type=jnp.float32)
    )


def _combine(agg2, x, W, W_self):
    NB = 2000
    return pl.pallas_call(
        _out_body,
        grid=(N // NB,),
        in_specs=[
            pl.BlockSpec((NC, NB, D), lambda i: (0, i, 0)),
            pl.BlockSpec((NB, D), lambda i: (i, 0)),
            pl.BlockSpec((D, D), lambda i: (0, 0)),
            pl.BlockSpec((D, D), lambda i: (0, 0)),
        ],
        out_specs=pl.BlockSpec((NB, D), lambda i: (i, 0)),
        out_shape=jax.ShapeDtypeStruct((N, D), jnp.float32),
    )(agg2, x, W, W_self)


def kernel(x, edge_index, edge_type, rel_emb, W, W_self):
    src = edge_index[0].astype(jnp.int32)
    dst = edge_index[1].astype(jnp.int32)
    et = edge_type.astype(jnp.int32)

    table = _build_table(x, rel_emb)
    ci2 = _build_ci(src.reshape(E // D, D), et.reshape(E // D, D))

    ci_sl = ci2.reshape(NW, NPH, CPP, CHUNK)
    dst_sl = dst.reshape(NW, NPH, CPP, CHUNK)

    dum = jnp.zeros((CHUNK, D), jnp.float32)
    agg2 = _sc_agg(table, ci_sl, dst_sl, dum).reshape(NC, N, D)
    return _combine(agg2, x, W, W_self)
